# baseline (speedup 1.0000x reference)
import functools
import math

import jax
import jax.numpy as jnp
from jax import lax
from jax.experimental import pallas as pl
from jax.experimental.pallas import tpu as pltpu
from jax.experimental.pallas import tpu_sc as plsc

_SMOOTHING = 0.1
_PAD = 1

_NC = 2
_NS = 16
_NW = _NC * _NS


def _sc_gather_body(lp_flat, tgt_hbm, vt_hbm, tgt_v, idx_v, val_v, sem, *, n, v):
    b = n // _NW
    wid = lax.axis_index("s") * _NC + lax.axis_index("c")
    base = wid * b
    pltpu.sync_copy(tgt_hbm.at[pl.ds(base, b)], tgt_v)
    for j in range(b // 16):
        t16 = tgt_v[pl.ds(j * 16, 16)]
        i16 = lax.iota(jnp.int32, 16) + (base + j * 16)
        idx_v[pl.ds(j * 16, 16)] = i16 * v + t16
    pltpu.async_copy(lp_flat.at[idx_v], val_v, sem).wait()
    pltpu.sync_copy(val_v, vt_hbm.at[pl.ds(base, b)])


def _tc_body(tgt_ref, vt_ref, lp_ref, out_ref, *, eps, conf, c):
    pid = pl.program_id(0)
    blk = lp_ref[...]
    t = tgt_ref[...]
    vt = vt_ref[...]
    rowsum = jnp.sum(blk, axis=1, keepdims=True)
    vb = blk[:, _PAD:_PAD + 1]
    contrib = jnp.where(
        t != _PAD, c - eps * rowsum + eps * vb + (eps - conf) * vt, 0.0
    )
    s = jnp.sum(contrib)

    @pl.when(pid == 0)
    def _():
        out_ref[0, 0] = 0.0

    out_ref[0, 0] += s


def kernel(log_probs, targets):
    lp = log_probs.reshape(-1, log_probs.shape[-1])
    n, v = lp.shape
    tgt = targets.reshape(-1).astype(jnp.int32)

    sc_gather = pl.kernel(
        functools.partial(_sc_gather_body, n=n, v=v),
        out_type=jax.ShapeDtypeStruct((n,), jnp.float32),
        mesh=plsc.VectorSubcoreMesh(core_axis_name="c", subcore_axis_name="s"),
        scratch_types=[
            pltpu.VMEM((n // _NW,), jnp.int32),
            pltpu.VMEM((n // _NW,), jnp.int32),
            pltpu.VMEM((n // _NW,), jnp.float32),
            pltpu.SemaphoreType.DMA,
        ],
    )
    vt = sc_gather(lp.reshape(-1), tgt)

    rb = 64
    eps = _SMOOTHING / (v - 2)
    conf = 1.0 - _SMOOTHING
    c = (v - 2) * eps * math.log(eps) + conf * math.log(conf)
    out = pl.pallas_call(
        functools.partial(_tc_body, eps=eps, conf=conf, c=c),
        grid=(n // rb,),
        in_specs=[
            pl.BlockSpec((rb, 1), lambda i: (i, 0)),
            pl.BlockSpec((rb, 1), lambda i: (i, 0)),
            pl.BlockSpec((rb, v), lambda i: (i, 0)),
        ],
        out_specs=pl.BlockSpec(
            (1, 1), lambda i: (0, 0), memory_space=pltpu.SMEM
        ),
        out_shape=jax.ShapeDtypeStruct((1, 1), jnp.float32),
    )(tgt.reshape(n, 1), vt.reshape(n, 1), lp)
    return out[0, 0]


# speedup vs baseline: 2.2131x; 2.2131x over previous
import functools
import math

import jax
import jax.numpy as jnp
from jax import lax
from jax.experimental import pallas as pl
from jax.experimental.pallas import tpu as pltpu

_SMOOTHING = 0.1
_PAD = 1


def _body(tgt_ref, lp_ref, out_ref, *, eps, conf, c):
    pid = pl.program_id(0)
    blk = lp_ref[...]
    t = tgt_ref[...]
    rowsum = jnp.sum(blk, axis=1, keepdims=True)
    vb = blk[:, _PAD:_PAD + 1]
    cols = lax.broadcasted_iota(jnp.int32, blk.shape, 1)
    vt = jnp.sum(jnp.where(cols == t, blk, 0.0), axis=1, keepdims=True)
    contrib = jnp.where(
        t != _PAD, c - eps * rowsum + eps * vb + (eps - conf) * vt, 0.0
    )
    s = jnp.sum(contrib)

    @pl.when(pid == 0)
    def _():
        out_ref[0, 0] = 0.0

    out_ref[0, 0] += s


def kernel(log_probs, targets):
    lp = log_probs.reshape(-1, log_probs.shape[-1])
    n, v = lp.shape
    tgt = targets.reshape(-1, 1).astype(jnp.int32)
    rb = 32
    eps = _SMOOTHING / (v - 2)
    conf = 1.0 - _SMOOTHING
    c = (v - 2) * eps * math.log(eps) + conf * math.log(conf)
    out = pl.pallas_call(
        functools.partial(_body, eps=eps, conf=conf, c=c),
        grid=(n // rb,),
        in_specs=[
            pl.BlockSpec((rb, 1), lambda i: (i, 0)),
            pl.BlockSpec((rb, v), lambda i: (i, 0)),
        ],
        out_specs=pl.BlockSpec(
            (1, 1), lambda i: (0, 0), memory_space=pltpu.SMEM
        ),
        out_shape=jax.ShapeDtypeStruct((1, 1), jnp.float32),
    )(tgt, lp)
    return out[0, 0]
